# baseline (speedup 1.0000x reference)
import functools

import jax
import jax.numpy as jnp
from jax import lax
from jax.experimental import pallas as pl
from jax.experimental.pallas import tpu as pltpu
from jax.experimental.pallas import tpu_sc as plsc

_NUM_CORES = 2
_NUM_SUBCORES = 16
_NW = _NUM_CORES * _NUM_SUBCORES
_BATCH = 16384
_DIM = 64
_BPW = _BATCH // _NW
_CHUNK = 128
_NCHUNK = _BPW // _CHUNK

_mesh = plsc.VectorSubcoreMesh(core_axis_name="c", subcore_axis_name="s")


@functools.partial(
    pl.kernel,
    mesh=_mesh,
    out_type=(
        jax.ShapeDtypeStruct((_BATCH, _DIM), jnp.float32),
        jax.ShapeDtypeStruct((_BATCH, _DIM), jnp.float32),
    ),
    scratch_types=[
        pltpu.VMEM((_BPW,), jnp.int32),
        pltpu.VMEM((_BPW, _DIM), jnp.float32),
        pltpu.VMEM((_BPW,), jnp.int32),
        pltpu.VMEM((_BPW, _DIM), jnp.float32),
        pltpu.SemaphoreType.DMA,
        pltpu.SemaphoreType.DMA,
    ],
    compiler_params=pltpu.CompilerParams(use_tc_tiling_on_sc=False),
)
def _gather2(user_hbm, item_hbm, uidx_hbm, iidx_hbm, u_out, i_out,
             uidx_v, urows_v, iidx_v, irows_v, usem, isem):
    wid = lax.axis_index("s") * _NUM_CORES + lax.axis_index("c")
    base = wid * _BPW
    pltpu.sync_copy(uidx_hbm.at[pl.ds(base, _BPW)], uidx_v)
    pltpu.sync_copy(iidx_hbm.at[pl.ds(base, _BPW)], iidx_v)
    ucopies = []
    icopies = []
    for c in range(_NCHUNK):
        sl = pl.ds(c * _CHUNK, _CHUNK)
        ucopies.append(
            pltpu.async_copy(user_hbm.at[uidx_v.at[sl]], urows_v.at[sl], usem))
        icopies.append(
            pltpu.async_copy(item_hbm.at[iidx_v.at[sl]], irows_v.at[sl], isem))
    for cp in ucopies:
        cp.wait()
    pltpu.sync_copy(urows_v, u_out.at[pl.ds(base, _BPW)])
    for cp in icopies:
        cp.wait()
    pltpu.sync_copy(irows_v, i_out.at[pl.ds(base, _BPW)])


def kernel(user_emb, item_emb, user_indices, item_indices):
    return _gather2(user_emb, item_emb, user_indices, item_indices)
